# baseline (speedup 1.0000x reference)
import functools

import jax
import jax.numpy as jnp
from jax import lax
from jax.experimental import pallas as pl
from jax.experimental.pallas import tpu as pltpu
from jax.experimental.pallas import tpu_sc as plsc

F = 26
D = 64
B = 4096
FIELD_DIM = 38462
NC, NS = 2, 16
NW = NC * NS
ROWS = B * F
CHUNK = 128
CHUNKS_PER_W = ROWS // (NW * CHUNK)


def _sc_gather_body(table, idx_hbm, out, idx_v, rows_v, gsem):
    c = lax.axis_index("c")
    s = lax.axis_index("s")
    wid = s * NC + c
    base = wid * CHUNKS_PER_W
    pltpu.sync_copy(idx_hbm.at[wid], idx_v)

    def chunk(j, carry):
        pltpu.async_copy(table.at[idx_v.at[j]], rows_v.at[0], gsem).wait()
        pltpu.sync_copy(rows_v.at[0], out.at[pl.ds((base + j) * CHUNK, CHUNK)])
        return carry

    lax.fori_loop(0, CHUNKS_PER_W, chunk, 0)


def _sc_gather(table, idx2d):
    mesh = plsc.VectorSubcoreMesh(
        core_axis_name="c", subcore_axis_name="s", num_cores=NC, num_subcores=NS
    )
    return pl.kernel(
        _sc_gather_body,
        out_type=jax.ShapeDtypeStruct((ROWS, D), jnp.float32),
        mesh=mesh,
        scratch_types=[
            pltpu.VMEM((CHUNKS_PER_W, CHUNK), jnp.int32),
            pltpu.VMEM((1, CHUNK, D), jnp.float32),
            pltpu.SemaphoreType.DMA,
        ],
        compiler_params=pltpu.CompilerParams(use_tc_tiling_on_sc=False),
        name="sc_emb_gather",
    )(table, idx2d)


def _mlp_body(x_ref, w1, b1, g1, be1, rm1, rv1, w2, b2, g2, be2, rm2, rv2,
              w3, b3, g3, be3, rm3, rv3, wo, bo, out_ref):
    h = x_ref[...]
    for (w, b, g, be, rm, rv) in ((w1, b1, g1, be1, rm1, rv1),
                                  (w2, b2, g2, be2, rm2, rv2),
                                  (w3, b3, g3, be3, rm3, rv3)):
        h = lax.dot_general(h, w[...], (((1,), (1,)), ((), ())),
                            preferred_element_type=jnp.float32)
        scale = g[...] * lax.rsqrt(rv[...] + 1e-5)
        h = (h + (b[...] - rm[...])) * scale + be[...]
        h = jnp.maximum(h, 0.0)
    o = lax.dot_general(wo[...], h, (((1,), (1,)), ((), ())),
                        preferred_element_type=jnp.float32)
    out_ref[...] = jax.nn.sigmoid(o + bo[...])[0]


def _tc_mlp(x, w1, b1, g1, be1, rm1, rv1, w2, b2, g2, be2, rm2, rv2,
            w3, b3, g3, be3, rm3, rv3, wo, bo):
    blk = 512
    grid = (B // blk,)
    full = lambda shape: pl.BlockSpec(shape, lambda m: (0,) * len(shape))
    in_specs = [
        pl.BlockSpec((blk, F * D), lambda m: (m, 0)),
        full((1024, F * D)), full((1024,)), full((1024,)), full((1024,)),
        full((1024,)), full((1024,)),
        full((512, 1024)), full((512,)), full((512,)), full((512,)),
        full((512,)), full((512,)),
        full((256, 512)), full((256,)), full((256,)), full((256,)),
        full((256,)), full((256,)),
        full((1, 256)), full((1, 1)),
    ]
    out = pl.pallas_call(
        _mlp_body,
        grid=grid,
        in_specs=in_specs,
        out_specs=pl.BlockSpec((blk,), lambda m: (m,)),
        out_shape=jax.ShapeDtypeStruct((B,), jnp.float32),
        name="tc_mlp",
    )(x, w1, b1, g1, be1, rm1, rv1, w2, b2, g2, be2, rm2, rv2,
      w3, b3, g3, be3, rm3, rv3, wo, bo.reshape(1, 1))
    return out


def kernel(field, emb_table, W1, b1, g1, be1, rm1, rv1, W2, b2, g2, be2,
           rm2, rv2, W3, b3, g3, be3, rm3, rv3, Wo, bo):
    offsets = jnp.arange(F, dtype=jnp.int32) * FIELD_DIM
    idx = (field + offsets[None, :]).reshape(NW, CHUNKS_PER_W, CHUNK)
    gathered = _sc_gather(emb_table, idx)
    x = gathered.reshape(B, F * D)
    W1p = W1.reshape(1024, D, F).transpose(0, 2, 1).reshape(1024, F * D)
    return _tc_mlp(x, W1p, b1, g1, be1, rm1, rv1, W2, b2, g2, be2, rm2, rv2,
                   W3, b3, g3, be3, rm3, rv3, Wo, bo)
